# Initial kernel scaffold; baseline (speedup 1.0000x reference)
import jax
import jax.numpy as jnp
from jax.experimental import pallas as pl


def kernel(x, edge_index, attention, W, b):
    raise NotImplementedError("write your pallas kernel here")


# speedup vs baseline: 2.7299x; 2.7299x over previous
import functools

import jax
import jax.numpy as jnp
from jax import lax
from jax.experimental import pallas as pl
from jax.experimental.pallas import tpu as pltpu
from jax.experimental.pallas import tpu_sc as plsc

N_NODES = 100000
N_EDGES = 100000
D = 256
H = 4

NC = 2
NS = 16
L = 16
NW = NC * NS

CHUNK = 3200
EPAD = NW * CHUNK
SUB = 128
NSUB = CHUNK // SUB
NP2 = 50176
NPT2 = NP2 // NS
NRO = NPT2 // 8

_mesh = plsc.VectorSubcoreMesh(core_axis_name="c", subcore_axis_name="s")
_sc_params = pltpu.CompilerParams(
    needs_layout_passes=False, use_tc_tiling_on_sc=False)


_B1 = 2000


def _k1_body(x_ref, m_ref, o_ref):
    o_ref[...] = jnp.dot(x_ref[...], m_ref[...],
                         preferred_element_type=jnp.float32)


_k1 = pl.pallas_call(
    _k1_body,
    grid=(N_NODES // _B1,),
    in_specs=[
        pl.BlockSpec((_B1, D), lambda i: (i, 0)),
        pl.BlockSpec((D, 2 * H), lambda i: (0, 0)),
    ],
    out_specs=pl.BlockSpec((_B1, 2 * H), lambda i: (i, 0)),
    out_shape=jax.ShapeDtypeStruct((N_NODES, 2 * H), jnp.float32),
)


@functools.partial(
    pl.kernel,
    out_type=(
        jax.ShapeDtypeStruct((EPAD, H), jnp.float32),
        jax.ShapeDtypeStruct((2 * NP2, 2 * H), jnp.float32),
    ),
    mesh=_mesh,
    compiler_params=_sc_params,
    scratch_types=[
        pltpu.VMEM((NSUB, SUB), jnp.int32),
        pltpu.VMEM((NSUB, SUB), jnp.int32),
        pltpu.VMEM((NSUB, SUB), jnp.int32),
        pltpu.VMEM((SUB, 2 * H), jnp.float32),
        pltpu.VMEM((SUB, 2 * H), jnp.float32),
        pltpu.VMEM((CHUNK, H), jnp.float32),
        pltpu.VMEM((CHUNK, 2 * H), jnp.float32),
        pltpu.VMEM((NRO, 2 * H), jnp.float32),
        pltpu.VMEM_SHARED((NP2, 2 * H), jnp.float32),
        pltpu.SemaphoreType.DMA,
    ],
)
def _k2(pq_hbm, row_hbm, col_hbm, zeros_hbm, g2_hbm, z_hbm,
        idx_r, idx_c, idx_h, prb, qcb, g2v, g2v8, znode, zsh, sem):
    cid = lax.axis_index("c")
    sid = lax.axis_index("s")
    wid = cid * NS + sid
    base = pl.multiple_of(wid * CHUNK, SUB)
    nb = pl.multiple_of(sid * NPT2, 8)

    def _zero(k, carry):
        off = pl.multiple_of(nb + k * NRO, 8)
        pltpu.sync_copy(zeros_hbm.at[pl.ds(off, NRO)], znode)
        pltpu.sync_copy(znode, zsh.at[pl.ds(off, NRO)])
        return carry

    lax.fori_loop(0, 8, _zero, 0)
    pltpu.sync_copy(zeros_hbm.at[pl.ds(0, CHUNK)], g2v8)
    plsc.subcore_barrier()

    iot = jnp.arange(L, dtype=jnp.int32)

    def _batch(i, carry):
        off = pl.multiple_of(base + i * SUB, SUB)
        pltpu.sync_copy(row_hbm.at[pl.ds(off, SUB)], idx_r.at[i])
        pltpu.sync_copy(col_hbm.at[pl.ds(off, SUB)], idx_c.at[i])
        pltpu.async_copy(pq_hbm.at[idx_r.at[i]], prb, sem).wait()
        pltpu.async_copy(pq_hbm.at[idx_c.at[i]], qcb, sem).wait()
        ivec = jnp.zeros((L,), jnp.int32) + i

        def _compute(j, c2):
            lvec = j * L + iot
            evec = i * SUB + lvec
            s = []
            for h in range(H):
                a = plsc.load_gather(prb, [lvec, jnp.full((L,), h, jnp.int32)])
                q = plsc.load_gather(qcb, [lvec, jnp.full((L,), H + h, jnp.int32)])
                t = a + q
                s.append(jnp.where(t >= 0.0, t, 0.01 * t))
            m = jnp.maximum(jnp.maximum(s[0], s[1]), jnp.maximum(s[2], s[3]))
            e = [jnp.exp(sh - m) for sh in s]
            den = (e[0] + e[1]) + (e[2] + e[3])
            valid = (base + evec) < N_EDGES
            rv = plsc.load_gather(idx_r, [ivec, lvec])
            plsc.store_scatter(idx_h, [ivec, lvec],
                               lax.shift_right_logical(rv, 1))
            halfoff = (rv & 1) * H
            for h in range(H):
                g2h = jnp.exp(e[h] / den)
                g2h = jnp.where(valid, g2h, 0.0)
                plsc.store_scatter(
                    g2v, [evec, jnp.full((L,), h, jnp.int32)], g2h)
                plsc.store_scatter(g2v8, [evec, halfoff + h], g2h)
            return c2

        lax.fori_loop(0, SUB // L, _compute, 0)
        return carry

    lax.fori_loop(0, NSUB, _batch, 0)

    pltpu.sync_copy(g2v, g2_hbm.at[pl.ds(base, CHUNK)])

    def _scat(i, carry):
        pltpu.sync_copy(g2v8.at[pl.ds(i * SUB, SUB)],
                        zsh.at[idx_h.at[i]], add=True)
        return carry

    lax.fori_loop(0, NSUB, _scat, 0)
    plsc.subcore_barrier()

    def _readout(k, carry):
        off = pl.multiple_of(nb + k * NRO, 8)
        dst = pl.multiple_of(cid * NP2 + off, 8)
        pltpu.sync_copy(zsh.at[pl.ds(off, NRO)], znode)
        pltpu.sync_copy(znode, z_hbm.at[pl.ds(dst, NRO)])
        return carry

    lax.fori_loop(0, 8, _readout, 0)


@functools.partial(
    pl.kernel,
    out_type=(
        jax.ShapeDtypeStruct((EPAD, 2 * H), jnp.float32),
        jax.ShapeDtypeStruct((EPAD, 2 * H), jnp.float32),
        jax.ShapeDtypeStruct((EPAD, D), jnp.float32),
    ),
    mesh=_mesh,
    compiler_params=_sc_params,
    scratch_types=[
        pltpu.VMEM((NSUB, SUB), jnp.int32),
        pltpu.VMEM((NSUB, SUB), jnp.int32),
        pltpu.VMEM((NSUB, SUB), jnp.int32),
        pltpu.VMEM((NSUB, SUB), jnp.int32),
        pltpu.VMEM((CHUNK, 2 * H), jnp.float32),
        pltpu.VMEM((CHUNK, 2 * H), jnp.float32),
        pltpu.VMEM((SUB, D), jnp.float32),
        pltpu.SemaphoreType.DMA,
        pltpu.SemaphoreType.DMA,
    ],
)
def _k3(row_hbm, col_hbm, z_hbm, x_hbm, zp0_hbm, zp1_hbm, xc_hbm,
        idx_r, idx_c, idx_h, idx_h2, zp0v, zp1v, xbuf, sem0, sem1):
    cid = lax.axis_index("c")
    sid = lax.axis_index("s")
    wid = cid * NS + sid
    base = pl.multiple_of(wid * CHUNK, SUB)

    iot = jnp.arange(L, dtype=jnp.int32)

    def _load(i, carry):
        off = pl.multiple_of(base + i * SUB, SUB)
        pltpu.sync_copy(row_hbm.at[pl.ds(off, SUB)], idx_r.at[i])
        pltpu.sync_copy(col_hbm.at[pl.ds(off, SUB)], idx_c.at[i])
        ivec = jnp.zeros((L,), jnp.int32) + i

        def _half(j, c2):
            lvec = j * L + iot
            rv = plsc.load_gather(idx_r, [ivec, lvec])
            hv = lax.shift_right_logical(rv, 1)
            plsc.store_scatter(idx_h, [ivec, lvec], hv)
            plsc.store_scatter(idx_h2, [ivec, lvec], hv + NP2)
            return c2

        lax.fori_loop(0, SUB // L, _half, 0)
        return carry

    lax.fori_loop(0, NSUB, _load, 0)

    def _gz(i, carry):
        pltpu.async_copy(z_hbm.at[idx_h.at[i]],
                         zp0v.at[pl.ds(i * SUB, SUB)], sem0).wait()
        pltpu.async_copy(z_hbm.at[idx_h2.at[i]],
                         zp1v.at[pl.ds(i * SUB, SUB)], sem0).wait()
        return carry

    lax.fori_loop(0, NSUB, _gz, 0)
    pltpu.sync_copy(zp0v, zp0_hbm.at[pl.ds(base, CHUNK)])
    pltpu.sync_copy(zp1v, zp1_hbm.at[pl.ds(base, CHUNK)])

    def _gx(i, carry):
        pltpu.async_copy(x_hbm.at[idx_c.at[i]], xbuf, sem1).wait()
        off = pl.multiple_of(base + i * SUB, SUB)
        pltpu.sync_copy(xbuf, xc_hbm.at[pl.ds(off, SUB)])
        return carry

    lax.fori_loop(0, NSUB, _gx, 0)


_B4 = 800


def _k4_body(xc_ref, g2_ref, zp0_ref, zp1_ref, par_ref, x_ref, wt_ref,
             b_ref, o_ref):
    zs = zp0_ref[...] + zp1_ref[...]
    par = par_ref[...]
    den = (1.0 - par) * zs[:, :H] + par * zs[:, H:]
    w = g2_ref[...] / den
    xc = xc_ref[...]
    hcat = jnp.concatenate(
        [xc * w[:, h:h + 1] for h in range(H)], axis=1)
    acc = jnp.dot(hcat, wt_ref[...], preferred_element_type=jnp.float32)
    o_ref[...] = acc + x_ref[...] + b_ref[...]


_k4 = pl.pallas_call(
    _k4_body,
    grid=(N_EDGES // _B4,),
    in_specs=[
        pl.BlockSpec((_B4, D), lambda i: (i, 0)),
        pl.BlockSpec((_B4, H), lambda i: (i, 0)),
        pl.BlockSpec((_B4, 2 * H), lambda i: (i, 0)),
        pl.BlockSpec((_B4, 2 * H), lambda i: (i, 0)),
        pl.BlockSpec((_B4, 1), lambda i: (i, 0)),
        pl.BlockSpec((_B4, D), lambda i: (i, 0)),
        pl.BlockSpec((H * D, D), lambda i: (0, 0)),
        pl.BlockSpec((1, D), lambda i: (0, 0)),
    ],
    out_specs=pl.BlockSpec((_B4, D), lambda i: (i, 0)),
    out_shape=jax.ShapeDtypeStruct((N_EDGES, D), jnp.float32),
)


def kernel(x, edge_index, attention, W, b):
    att = attention[0]
    M = jnp.concatenate([att[:, :D].T, att[:, D:].T], axis=1)
    Wt = W.T
    b2 = b.reshape(1, D)
    rowp = jnp.pad(edge_index[0], (0, EPAD - N_EDGES))
    colp = jnp.pad(edge_index[1], (0, EPAD - N_EDGES))
    parf = (rowp & 1).astype(jnp.float32).reshape(EPAD, 1)
    zeros = jnp.zeros((NP2, 2 * H), jnp.float32)

    pq = _k1(x, M)
    g2, z = _k2(pq, rowp, colp, zeros)
    zp0, zp1, xc = _k3(rowp, colp, z, x)
    return _k4(xc, g2, zp0, zp1, parf, x, Wt, b2)


# speedup vs baseline: 3.8030x; 1.3931x over previous
import functools

import jax
import jax.numpy as jnp
from jax import lax
from jax.experimental import pallas as pl
from jax.experimental.pallas import tpu as pltpu
from jax.experimental.pallas import tpu_sc as plsc

N_NODES = 100000
N_EDGES = 100000
D = 256
H = 4

NC = 2
NS = 16
L = 16
NW = NC * NS

CHUNK = 3200
EPAD = NW * CHUNK
SUB = 128
NSUB = CHUNK // SUB
NP2 = 50176
NPT2 = NP2 // NS
NRO = NPT2 // 16
NB = 3

_mesh = plsc.VectorSubcoreMesh(core_axis_name="c", subcore_axis_name="s")
_sc_params = pltpu.CompilerParams(
    needs_layout_passes=False, use_tc_tiling_on_sc=False)
_sc_params_tc_tiled = pltpu.CompilerParams(needs_layout_passes=False)


_B1 = 2000


def _k1_body(x_ref, m_ref, o_ref):
    o_ref[...] = jnp.dot(x_ref[...], m_ref[...],
                         preferred_element_type=jnp.float32)


_k1 = pl.pallas_call(
    _k1_body,
    grid=(N_NODES // _B1,),
    in_specs=[
        pl.BlockSpec((_B1, D), lambda i: (i, 0)),
        pl.BlockSpec((D, 2 * H), lambda i: (0, 0)),
    ],
    out_specs=pl.BlockSpec((_B1, 2 * H), lambda i: (i, 0)),
    out_shape=jax.ShapeDtypeStruct((N_NODES, 2 * H), jnp.float32),
)


@functools.partial(
    pl.kernel,
    out_type=(
        jax.ShapeDtypeStruct((EPAD, 2 * H), jnp.float32),
        jax.ShapeDtypeStruct((2 * NP2, 2 * H), jnp.float32),
    ),
    mesh=_mesh,
    compiler_params=_sc_params,
    scratch_types=[
        pltpu.VMEM((CHUNK,), jnp.int32),
        pltpu.VMEM((CHUNK,), jnp.int32),
        pltpu.VMEM((NSUB, SUB), jnp.int32),
        pltpu.VMEM((CHUNK, 2 * H), jnp.float32),
        pltpu.VMEM((CHUNK, 2 * H), jnp.float32),
        pltpu.VMEM((CHUNK, 2 * H), jnp.float32),
        pltpu.VMEM((NRO, 2 * H), jnp.float32),
        pltpu.VMEM_SHARED((NP2, 2 * H), jnp.float32),
        pltpu.SemaphoreType.DMA,
        pltpu.SemaphoreType.DMA,
    ],
)
def _k2(pq_hbm, row_hbm, col_hbm, zeros_hbm, g2_hbm, z_hbm,
        idx_r, idx_c, idx_h, pr, qc, g2v8, znode, zsh, sem, sem2):
    cid = lax.axis_index("c")
    sid = lax.axis_index("s")
    wid = cid * NS + sid
    base = pl.multiple_of(wid * CHUNK, SUB)
    nb = pl.multiple_of(sid * NPT2, 8)

    def _zero(k, carry):
        off = pl.multiple_of(nb + k * NRO, 8)
        pltpu.sync_copy(zeros_hbm.at[pl.ds(off, NRO)], znode)
        pltpu.sync_copy(znode, zsh.at[pl.ds(off, NRO)])
        return carry

    lax.fori_loop(0, 16, _zero, 0)
    pltpu.sync_copy(zeros_hbm.at[pl.ds(0, CHUNK)], g2v8)

    pltpu.sync_copy(row_hbm.at[pl.ds(base, CHUNK)], idx_r)
    pltpu.sync_copy(col_hbm.at[pl.ds(base, CHUNK)], idx_c)

    def _fire(i, carry):
        sl = pl.ds(i * SUB, SUB)
        pltpu.async_copy(pq_hbm.at[idx_r.at[sl]], pr.at[sl], sem)
        pltpu.async_copy(pq_hbm.at[idx_c.at[sl]], qc.at[sl], sem)
        return carry

    def _drain(i, carry):
        sl = pl.ds(i * SUB, SUB)
        pltpu.make_async_copy(pq_hbm.at[idx_r.at[sl]], pr.at[sl], sem).wait()
        pltpu.make_async_copy(pq_hbm.at[idx_c.at[sl]], qc.at[sl], sem).wait()
        return carry

    lax.fori_loop(0, NSUB, _fire, 0)
    plsc.subcore_barrier()
    lax.fori_loop(0, NSUB, _drain, 0)

    iot = jnp.arange(L, dtype=jnp.int32)

    def _compute(j, carry):
        evec = j * L + iot
        s = []
        for h in range(H):
            a = plsc.load_gather(pr, [evec, jnp.full((L,), h, jnp.int32)])
            q = plsc.load_gather(qc, [evec, jnp.full((L,), H + h, jnp.int32)])
            t = a + q
            s.append(jnp.where(t >= 0.0, t, 0.01 * t))
        m = jnp.maximum(jnp.maximum(s[0], s[1]), jnp.maximum(s[2], s[3]))
        e = [jnp.exp(sh - m) for sh in s]
        den = (e[0] + e[1]) + (e[2] + e[3])
        valid = (base + evec) < N_EDGES
        rv = plsc.load_gather(idx_r, [evec])
        plsc.store_scatter(idx_h, [lax.shift_right_logical(evec, 7),
                                   evec & (SUB - 1)],
                           lax.shift_right_logical(rv, 1))
        halfoff = (rv & 1) * H
        for h in range(H):
            g2h = jnp.exp(e[h] / den)
            g2h = jnp.where(valid, g2h, 0.0)
            plsc.store_scatter(g2v8, [evec, halfoff + h], g2h)
        return carry

    lax.fori_loop(0, CHUNK // L, _compute, 0)

    pltpu.async_copy(g2v8, g2_hbm.at[pl.ds(base, CHUNK)], sem2)

    def _scat_fire(i, carry):
        pltpu.async_copy(g2v8.at[pl.ds(i * SUB, SUB)],
                         zsh.at[idx_h.at[i]], sem, add=True)
        return carry

    def _scat_drain(i, carry):
        pltpu.make_async_copy(g2v8.at[pl.ds(i * SUB, SUB)],
                              zsh.at[idx_h.at[i]], sem).wait()
        return carry

    lax.fori_loop(0, NSUB, _scat_fire, 0)
    lax.fori_loop(0, NSUB, _scat_drain, 0)
    pltpu.make_async_copy(g2v8, g2_hbm.at[pl.ds(base, CHUNK)], sem2).wait()
    plsc.subcore_barrier()

    def _readout(k, carry):
        off = pl.multiple_of(nb + k * NRO, 8)
        dst = pl.multiple_of(cid * NP2 + off, 8)
        pltpu.sync_copy(zsh.at[pl.ds(off, NRO)], znode)
        pltpu.sync_copy(znode, z_hbm.at[pl.ds(dst, NRO)])
        return carry

    lax.fori_loop(0, 16, _readout, 0)


@functools.partial(
    pl.kernel,
    out_type=(
        jax.ShapeDtypeStruct((EPAD, 2 * H), jnp.float32),
        jax.ShapeDtypeStruct((EPAD, 2 * H), jnp.float32),
    ),
    mesh=_mesh,
    compiler_params=_sc_params,
    scratch_types=[
        pltpu.VMEM((CHUNK,), jnp.int32),
        pltpu.VMEM((CHUNK,), jnp.int32),
        pltpu.VMEM((CHUNK,), jnp.int32),
        pltpu.VMEM((CHUNK, 2 * H), jnp.float32),
        pltpu.VMEM((CHUNK, 2 * H), jnp.float32),
        pltpu.SemaphoreType.DMA,
    ],
)
def _k3a(row_hbm, z_hbm, zp0_hbm, zp1_hbm,
         idx_r, idx_h1, idx_h2, zp0v, zp1v, sem):
    cid = lax.axis_index("c")
    sid = lax.axis_index("s")
    wid = cid * NS + sid
    base = pl.multiple_of(wid * CHUNK, SUB)

    pltpu.sync_copy(row_hbm.at[pl.ds(base, CHUNK)], idx_r)
    iot = jnp.arange(L, dtype=jnp.int32)

    def _half(j, carry):
        evec = j * L + iot
        rv = plsc.load_gather(idx_r, [evec])
        hv = lax.shift_right_logical(rv, 1)
        plsc.store_scatter(idx_h1, [evec], hv)
        plsc.store_scatter(idx_h2, [evec], hv + NP2)
        return carry

    lax.fori_loop(0, CHUNK // L, _half, 0)

    def _fire(i, carry):
        sl = pl.ds(i * SUB, SUB)
        pltpu.async_copy(z_hbm.at[idx_h1.at[sl]], zp0v.at[sl], sem)
        pltpu.async_copy(z_hbm.at[idx_h2.at[sl]], zp1v.at[sl], sem)
        return carry

    def _drain(i, carry):
        sl = pl.ds(i * SUB, SUB)
        pltpu.make_async_copy(z_hbm.at[idx_h1.at[sl]], zp0v.at[sl], sem).wait()
        pltpu.make_async_copy(z_hbm.at[idx_h2.at[sl]], zp1v.at[sl], sem).wait()
        return carry

    lax.fori_loop(0, NSUB, _fire, 0)
    lax.fori_loop(0, NSUB, _drain, 0)
    pltpu.sync_copy(zp0v, zp0_hbm.at[pl.ds(base, CHUNK)])
    pltpu.sync_copy(zp1v, zp1_hbm.at[pl.ds(base, CHUNK)])


@functools.partial(
    pl.kernel,
    out_type=jax.ShapeDtypeStruct((EPAD, D), jnp.float32),
    mesh=_mesh,
    compiler_params=_sc_params_tc_tiled,
    scratch_types=[
        pltpu.VMEM((CHUNK,), jnp.int32),
        pltpu.VMEM((NB, SUB, D), jnp.float32),
        pltpu.SemaphoreType.DMA,
        pltpu.SemaphoreType.DMA,
    ],
)
def _k3b(col_hbm, x_hbm, xc_hbm, idx_c, xbuf, gsem, wsem):
    cid = lax.axis_index("c")
    sid = lax.axis_index("s")
    wid = cid * NS + sid
    base = pl.multiple_of(wid * CHUNK, SUB)

    pltpu.sync_copy(col_hbm.at[pl.ds(base, CHUNK)], idx_c)

    def _g_src(i):
        return x_hbm.at[idx_c.at[pl.ds(i * SUB, SUB)]]

    def _w_dst(i):
        return xc_hbm.at[pl.ds(base + i * SUB, SUB)]

    for p in range(NB - 1):
        pltpu.async_copy(_g_src(p), xbuf.at[p], gsem)

    def _step(i, carry):
        @pl.when(i >= 1)
        def _():
            pltpu.make_async_copy(xbuf.at[(i - 1) % NB], _w_dst(i - 1),
                                  wsem).wait()

        @pl.when(i + NB - 1 < NSUB)
        def _():
            pltpu.async_copy(_g_src(i + NB - 1), xbuf.at[(i + NB - 1) % NB],
                             gsem)

        pltpu.make_async_copy(_g_src(i), xbuf.at[i % NB], gsem).wait()
        pltpu.async_copy(xbuf.at[i % NB], _w_dst(i), wsem)
        return carry

    lax.fori_loop(0, NSUB, _step, 0)
    pltpu.make_async_copy(xbuf.at[(NSUB - 1) % NB], _w_dst(NSUB - 1),
                          wsem).wait()


_B4 = 800


def _k4_body(xc_ref, g2_ref, zp0_ref, zp1_ref, par_ref, x_ref, wt_ref,
             b_ref, o_ref):
    zs = zp0_ref[...] + zp1_ref[...]
    par = par_ref[...]
    den = (1.0 - par) * zs[:, :H] + par * zs[:, H:]
    g2p = g2_ref[...]
    g2 = (1.0 - par) * g2p[:, :H] + par * g2p[:, H:]
    w = g2 / den
    xc = xc_ref[...]
    hcat = jnp.concatenate(
        [xc * w[:, h:h + 1] for h in range(H)], axis=1)
    acc = jnp.dot(hcat, wt_ref[...], preferred_element_type=jnp.float32)
    o_ref[...] = acc + x_ref[...] + b_ref[...]


_k4 = pl.pallas_call(
    _k4_body,
    grid=(N_EDGES // _B4,),
    in_specs=[
        pl.BlockSpec((_B4, D), lambda i: (i, 0)),
        pl.BlockSpec((_B4, 2 * H), lambda i: (i, 0)),
        pl.BlockSpec((_B4, 2 * H), lambda i: (i, 0)),
        pl.BlockSpec((_B4, 2 * H), lambda i: (i, 0)),
        pl.BlockSpec((_B4, 1), lambda i: (i, 0)),
        pl.BlockSpec((_B4, D), lambda i: (i, 0)),
        pl.BlockSpec((H * D, D), lambda i: (0, 0)),
        pl.BlockSpec((1, D), lambda i: (0, 0)),
    ],
    out_specs=pl.BlockSpec((_B4, D), lambda i: (i, 0)),
    out_shape=jax.ShapeDtypeStruct((N_EDGES, D), jnp.float32),
)


def kernel(x, edge_index, attention, W, b):
    att = attention[0]
    M = jnp.concatenate([att[:, :D].T, att[:, D:].T], axis=1)
    Wt = W.T
    b2 = b.reshape(1, D)
    rowp = jnp.pad(edge_index[0], (0, EPAD - N_EDGES))
    colp = jnp.pad(edge_index[1], (0, EPAD - N_EDGES))
    parf = (rowp & 1).astype(jnp.float32).reshape(EPAD, 1)
    zeros = jnp.zeros((NP2, 2 * H), jnp.float32)

    pq = _k1(x, M)
    g2, z = _k2(pq, rowp, colp, zeros)
    zp0, zp1 = _k3a(rowp, z)
    xc = _k3b(colp, x)
    return _k4(xc, g2, zp0, zp1, parf, x, Wt, b2)


# speedup vs baseline: 4.0515x; 1.0653x over previous
import functools

import jax
import jax.numpy as jnp
from jax import lax
from jax.experimental import pallas as pl
from jax.experimental.pallas import tpu as pltpu
from jax.experimental.pallas import tpu_sc as plsc

N_NODES = 100000
N_EDGES = 100000
D = 256
H = 4

NC = 2
NS = 16
L = 16
NW = NC * NS

CHUNK = 3200
EPAD = NW * CHUNK
SUB = 128
NSUB = CHUNK // SUB
NP2 = 50176
NPT2 = NP2 // NS
NRO = NPT2 // 16
NB = 3

_mesh = plsc.VectorSubcoreMesh(core_axis_name="c", subcore_axis_name="s")
_sc_params = pltpu.CompilerParams(
    needs_layout_passes=False, use_tc_tiling_on_sc=False)
_sc_params_tc_tiled = pltpu.CompilerParams(needs_layout_passes=False)


_B1 = 2000


def _rne16(b):
    return (b + 0x7FFF + ((b >> 16) & 1)) >> 16


def _k1_body(x_ref, m_ref, o_ref, xb_ref):
    xv = x_ref[...]
    o_ref[...] = jnp.dot(xv, m_ref[...],
                         preferred_element_type=jnp.float32)
    lo = _rne16(pltpu.bitcast(xv[:, :D // 2], jnp.uint32))
    hi = _rne16(pltpu.bitcast(xv[:, D // 2:], jnp.uint32))
    xb_ref[...] = lo | (hi << 16)


_k1 = pl.pallas_call(
    _k1_body,
    grid=(N_NODES // _B1,),
    in_specs=[
        pl.BlockSpec((_B1, D), lambda i: (i, 0)),
        pl.BlockSpec((D, 2 * H), lambda i: (0, 0)),
    ],
    out_specs=(
        pl.BlockSpec((_B1, 2 * H), lambda i: (i, 0)),
        pl.BlockSpec((_B1, D // 2), lambda i: (i, 0)),
    ),
    out_shape=(
        jax.ShapeDtypeStruct((N_NODES, 2 * H), jnp.float32),
        jax.ShapeDtypeStruct((N_NODES, D // 2), jnp.uint32),
    ),
)


@functools.partial(
    pl.kernel,
    out_type=(
        jax.ShapeDtypeStruct((EPAD, 2 * H), jnp.float32),
        jax.ShapeDtypeStruct((2 * NP2, 2 * H), jnp.float32),
    ),
    mesh=_mesh,
    compiler_params=_sc_params,
    scratch_types=[
        pltpu.VMEM((CHUNK,), jnp.int32),
        pltpu.VMEM((CHUNK,), jnp.int32),
        pltpu.VMEM((NSUB, SUB), jnp.int32),
        pltpu.VMEM((CHUNK, 2 * H), jnp.float32),
        pltpu.VMEM((CHUNK, 2 * H), jnp.float32),
        pltpu.VMEM((CHUNK, 2 * H), jnp.float32),
        pltpu.VMEM((NRO, 2 * H), jnp.float32),
        pltpu.VMEM_SHARED((NP2, 2 * H), jnp.float32),
        pltpu.SemaphoreType.DMA,
        pltpu.SemaphoreType.DMA,
    ],
)
def _k2(pq_hbm, row_hbm, col_hbm, zeros_hbm, g2_hbm, z_hbm,
        idx_r, idx_c, idx_h, pr, qc, g2v8, znode, zsh, sem, sem2):
    cid = lax.axis_index("c")
    sid = lax.axis_index("s")
    wid = cid * NS + sid
    base = pl.multiple_of(wid * CHUNK, SUB)
    nb = pl.multiple_of(sid * NPT2, 8)

    def _zero(k, carry):
        off = pl.multiple_of(nb + k * NRO, 8)
        pltpu.sync_copy(zeros_hbm.at[pl.ds(off, NRO)], znode)
        pltpu.sync_copy(znode, zsh.at[pl.ds(off, NRO)])
        return carry

    lax.fori_loop(0, 16, _zero, 0)
    pltpu.sync_copy(zeros_hbm.at[pl.ds(0, CHUNK)], g2v8)

    pltpu.sync_copy(row_hbm.at[pl.ds(base, CHUNK)], idx_r)
    pltpu.sync_copy(col_hbm.at[pl.ds(base, CHUNK)], idx_c)

    def _fire(i, carry):
        sl = pl.ds(i * SUB, SUB)
        pltpu.async_copy(pq_hbm.at[idx_r.at[sl]], pr.at[sl], sem)
        pltpu.async_copy(pq_hbm.at[idx_c.at[sl]], qc.at[sl], sem)
        return carry

    def _drain(i, carry):
        sl = pl.ds(i * SUB, SUB)
        pltpu.make_async_copy(pq_hbm.at[idx_r.at[sl]], pr.at[sl], sem).wait()
        pltpu.make_async_copy(pq_hbm.at[idx_c.at[sl]], qc.at[sl], sem).wait()
        return carry

    lax.fori_loop(0, NSUB, _fire, 0)
    plsc.subcore_barrier()
    lax.fori_loop(0, NSUB, _drain, 0)

    iot = jnp.arange(L, dtype=jnp.int32)

    def _compute(j, carry):
        evec = j * L + iot
        s = []
        for h in range(H):
            a = plsc.load_gather(pr, [evec, jnp.full((L,), h, jnp.int32)])
            q = plsc.load_gather(qc, [evec, jnp.full((L,), H + h, jnp.int32)])
            t = a + q
            s.append(jnp.where(t >= 0.0, t, 0.01 * t))
        m = jnp.maximum(jnp.maximum(s[0], s[1]), jnp.maximum(s[2], s[3]))
        e = [jnp.exp(sh - m) for sh in s]
        den = (e[0] + e[1]) + (e[2] + e[3])
        valid = (base + evec) < N_EDGES
        rv = plsc.load_gather(idx_r, [evec])
        plsc.store_scatter(idx_h, [lax.shift_right_logical(evec, 7),
                                   evec & (SUB - 1)],
                           lax.shift_right_logical(rv, 1))
        halfoff = (rv & 1) * H
        for h in range(H):
            g2h = jnp.exp(e[h] / den)
            g2h = jnp.where(valid, g2h, 0.0)
            plsc.store_scatter(g2v8, [evec, halfoff + h], g2h)
        return carry

    lax.fori_loop(0, CHUNK // L, _compute, 0)

    pltpu.async_copy(g2v8, g2_hbm.at[pl.ds(base, CHUNK)], sem2)

    def _scat_fire(i, carry):
        pltpu.async_copy(g2v8.at[pl.ds(i * SUB, SUB)],
                         zsh.at[idx_h.at[i]], sem, add=True)
        return carry

    def _scat_drain(i, carry):
        pltpu.make_async_copy(g2v8.at[pl.ds(i * SUB, SUB)],
                              zsh.at[idx_h.at[i]], sem).wait()
        return carry

    lax.fori_loop(0, NSUB, _scat_fire, 0)
    lax.fori_loop(0, NSUB, _scat_drain, 0)
    pltpu.make_async_copy(g2v8, g2_hbm.at[pl.ds(base, CHUNK)], sem2).wait()
    plsc.subcore_barrier()

    def _readout(k, carry):
        off = pl.multiple_of(nb + k * NRO, 8)
        dst = pl.multiple_of(cid * NP2 + off, 8)
        pltpu.sync_copy(zsh.at[pl.ds(off, NRO)], znode)
        pltpu.sync_copy(znode, z_hbm.at[pl.ds(dst, NRO)])
        return carry

    lax.fori_loop(0, 16, _readout, 0)


@functools.partial(
    pl.kernel,
    out_type=(
        jax.ShapeDtypeStruct((EPAD, 2 * H), jnp.float32),
        jax.ShapeDtypeStruct((EPAD, 2 * H), jnp.float32),
    ),
    mesh=_mesh,
    compiler_params=_sc_params,
    scratch_types=[
        pltpu.VMEM((CHUNK,), jnp.int32),
        pltpu.VMEM((CHUNK,), jnp.int32),
        pltpu.VMEM((CHUNK,), jnp.int32),
        pltpu.VMEM((CHUNK, 2 * H), jnp.float32),
        pltpu.VMEM((CHUNK, 2 * H), jnp.float32),
        pltpu.SemaphoreType.DMA,
    ],
)
def _k3a(row_hbm, z_hbm, zp0_hbm, zp1_hbm,
         idx_r, idx_h1, idx_h2, zp0v, zp1v, sem):
    cid = lax.axis_index("c")
    sid = lax.axis_index("s")
    wid = cid * NS + sid
    base = pl.multiple_of(wid * CHUNK, SUB)

    pltpu.sync_copy(row_hbm.at[pl.ds(base, CHUNK)], idx_r)
    iot = jnp.arange(L, dtype=jnp.int32)

    def _half(j, carry):
        evec = j * L + iot
        rv = plsc.load_gather(idx_r, [evec])
        hv = lax.shift_right_logical(rv, 1)
        plsc.store_scatter(idx_h1, [evec], hv)
        plsc.store_scatter(idx_h2, [evec], hv + NP2)
        return carry

    lax.fori_loop(0, CHUNK // L, _half, 0)

    def _fire(i, carry):
        sl = pl.ds(i * SUB, SUB)
        pltpu.async_copy(z_hbm.at[idx_h1.at[sl]], zp0v.at[sl], sem)
        pltpu.async_copy(z_hbm.at[idx_h2.at[sl]], zp1v.at[sl], sem)
        return carry

    def _drain(i, carry):
        sl = pl.ds(i * SUB, SUB)
        pltpu.make_async_copy(z_hbm.at[idx_h1.at[sl]], zp0v.at[sl], sem).wait()
        pltpu.make_async_copy(z_hbm.at[idx_h2.at[sl]], zp1v.at[sl], sem).wait()
        return carry

    lax.fori_loop(0, NSUB, _fire, 0)
    lax.fori_loop(0, NSUB, _drain, 0)
    pltpu.sync_copy(zp0v, zp0_hbm.at[pl.ds(base, CHUNK)])
    pltpu.sync_copy(zp1v, zp1_hbm.at[pl.ds(base, CHUNK)])


@functools.partial(
    pl.kernel,
    out_type=jax.ShapeDtypeStruct((EPAD, D // 2), jnp.uint32),
    mesh=_mesh,
    compiler_params=_sc_params_tc_tiled,
    scratch_types=[
        pltpu.VMEM((CHUNK,), jnp.int32),
        pltpu.VMEM((NB, SUB, D // 2), jnp.uint32),
        pltpu.SemaphoreType.DMA,
        pltpu.SemaphoreType.DMA,
    ],
)
def _k3b(col_hbm, x_hbm, xc_hbm, idx_c, xbuf, gsem, wsem):
    cid = lax.axis_index("c")
    sid = lax.axis_index("s")
    wid = cid * NS + sid
    base = pl.multiple_of(wid * CHUNK, SUB)

    pltpu.sync_copy(col_hbm.at[pl.ds(base, CHUNK)], idx_c)

    def _g_src(i):
        return x_hbm.at[idx_c.at[pl.ds(i * SUB, SUB)]]

    def _w_dst(i):
        return xc_hbm.at[pl.ds(base + i * SUB, SUB)]

    for p in range(NB - 1):
        pltpu.async_copy(_g_src(p), xbuf.at[p], gsem)

    def _step(i, carry):
        @pl.when(i >= 1)
        def _():
            pltpu.make_async_copy(xbuf.at[(i - 1) % NB], _w_dst(i - 1),
                                  wsem).wait()

        @pl.when(i + NB - 1 < NSUB)
        def _():
            pltpu.async_copy(_g_src(i + NB - 1), xbuf.at[(i + NB - 1) % NB],
                             gsem)

        pltpu.make_async_copy(_g_src(i), xbuf.at[i % NB], gsem).wait()
        pltpu.async_copy(xbuf.at[i % NB], _w_dst(i), wsem)
        return carry

    lax.fori_loop(0, NSUB, _step, 0)
    pltpu.make_async_copy(xbuf.at[(NSUB - 1) % NB], _w_dst(NSUB - 1),
                          wsem).wait()


_B4 = 800


def _k4_body(xc_ref, g2_ref, zp0_ref, zp1_ref, par_ref, x_ref, wt_ref,
             b_ref, o_ref):
    zs = zp0_ref[...] + zp1_ref[...]
    par = par_ref[...]
    den = (1.0 - par) * zs[:, :H] + par * zs[:, H:]
    g2p = g2_ref[...]
    g2 = (1.0 - par) * g2p[:, :H] + par * g2p[:, H:]
    w = g2 / den
    p = xc_ref[...]
    lo_f = pltpu.bitcast(p << 16, jnp.float32)
    hi_f = pltpu.bitcast(p & jnp.uint32(0xFFFF0000), jnp.float32)
    parts = []
    for h in range(H):
        wh = w[:, h:h + 1]
        parts.append((lo_f * wh).astype(jnp.bfloat16))
        parts.append((hi_f * wh).astype(jnp.bfloat16))
    hcat = jnp.concatenate(parts, axis=1)
    acc = jnp.dot(hcat, wt_ref[...], preferred_element_type=jnp.float32)
    o_ref[...] = acc + x_ref[...] + b_ref[...]


_k4 = pl.pallas_call(
    _k4_body,
    grid=(N_EDGES // _B4,),
    in_specs=[
        pl.BlockSpec((_B4, D // 2), lambda i: (i, 0)),
        pl.BlockSpec((_B4, 2 * H), lambda i: (i, 0)),
        pl.BlockSpec((_B4, 2 * H), lambda i: (i, 0)),
        pl.BlockSpec((_B4, 2 * H), lambda i: (i, 0)),
        pl.BlockSpec((_B4, 1), lambda i: (i, 0)),
        pl.BlockSpec((_B4, D), lambda i: (i, 0)),
        pl.BlockSpec((H * D, D), lambda i: (0, 0)),
        pl.BlockSpec((1, D), lambda i: (0, 0)),
    ],
    out_specs=pl.BlockSpec((_B4, D), lambda i: (i, 0)),
    out_shape=jax.ShapeDtypeStruct((N_EDGES, D), jnp.float32),
)


def kernel(x, edge_index, attention, W, b):
    att = attention[0]
    M = jnp.concatenate([att[:, :D].T, att[:, D:].T], axis=1)
    Wt = W.T.astype(jnp.bfloat16)
    b2 = b.reshape(1, D)
    rowp = jnp.pad(edge_index[0], (0, EPAD - N_EDGES))
    colp = jnp.pad(edge_index[1], (0, EPAD - N_EDGES))
    parf = (rowp & 1).astype(jnp.float32).reshape(EPAD, 1)
    zeros = jnp.zeros((NP2, 2 * H), jnp.float32)

    pq, xb = _k1(x, M)
    g2, z = _k2(pq, rowp, colp, zeros)
    zp0, zp1 = _k3a(rowp, z)
    xc = _k3b(colp, xb)
    return _k4(xc, g2, zp0, zp1, parf, x, Wt, b2)
